# baseline (speedup 1.0000x reference)
import jax
import jax.numpy as jnp
from jax.experimental import pallas as pl
from jax.experimental.pallas import tpu as pltpu

_NBUF = 2


def _body(x_hbm, h_ref, w_ref, o_hbm, s_ref, xbuf, obuf, insems, outsems):
    B = x_hbm.shape[0]
    s_ref[...] = h_ref[...][0][:, None, :] + w_ref[...][0][None, :, :]

    def in_copy(b):
        return pltpu.make_async_copy(
            x_hbm.at[b], xbuf.at[b % _NBUF], insems.at[b % _NBUF])

    def out_copy(b):
        return pltpu.make_async_copy(
            obuf.at[b % _NBUF], o_hbm.at[b], outsems.at[b % _NBUF])

    for b in range(min(_NBUF, B)):
        in_copy(b).start()
    for b in range(B):
        slot = b % _NBUF
        in_copy(b).wait()
        if b >= _NBUF:
            out_copy(b - _NBUF).wait()
        obuf[slot] = xbuf[slot] + s_ref[...]
        out_copy(b).start()
        if b + _NBUF < B:
            in_copy(b + _NBUF).start()
    for b in range(max(B - _NBUF, 0), B):
        out_copy(b).wait()


def kernel(x, h_table, w_table):
    B, H, W, D = x.shape
    return pl.pallas_call(
        _body,
        grid=(1,),
        in_specs=[
            pl.BlockSpec(memory_space=pl.ANY),
            pl.BlockSpec((1, H, D), lambda i: (0, 0, 0)),
            pl.BlockSpec((1, W, D), lambda i: (0, 0, 0)),
        ],
        out_specs=pl.BlockSpec(memory_space=pl.ANY),
        out_shape=jax.ShapeDtypeStruct((B, H, W, D), x.dtype),
        scratch_shapes=[
            pltpu.VMEM((H, W, D), x.dtype),
            pltpu.VMEM((_NBUF, H, W, D), x.dtype),
            pltpu.VMEM((_NBUF, H, W, D), x.dtype),
            pltpu.SemaphoreType.DMA((_NBUF,)),
            pltpu.SemaphoreType.DMA((_NBUF,)),
        ],
    )(x, h_table[None], w_table[None])


# speedup vs baseline: 1.0759x; 1.0759x over previous
import jax
import jax.numpy as jnp
from jax.experimental import pallas as pl
from jax.experimental.pallas import tpu as pltpu

_NBUF = 4
_CHUNK = 4


def _body(x_hbm, h_ref, w_ref, o_hbm, s_ref, xbuf, insems, outsems):
    B = x_hbm.shape[0]
    n = B // _CHUNK
    s_ref[...] = h_ref[...][0][:, None, :] + w_ref[...][0][None, :, :]

    def in_copy(i):
        return pltpu.make_async_copy(
            x_hbm.at[pl.ds(i * _CHUNK, _CHUNK)], xbuf.at[i % _NBUF],
            insems.at[i % _NBUF])

    def out_copy(i):
        return pltpu.make_async_copy(
            xbuf.at[i % _NBUF], o_hbm.at[pl.ds(i * _CHUNK, _CHUNK)],
            outsems.at[i % _NBUF])

    for i in range(min(_NBUF, n)):
        in_copy(i).start()
    for i in range(n):
        slot = i % _NBUF
        in_copy(i).wait()
        xbuf[slot] = xbuf[slot] + s_ref[...][None]
        out_copy(i).start()
        nxt = i + _NBUF
        if nxt < n:
            out_copy(i).wait()
            in_copy(nxt).start()
    for i in range(max(n - _NBUF, 0), n):
        out_copy(i).wait()


def kernel(x, h_table, w_table):
    B, H, W, D = x.shape
    return pl.pallas_call(
        _body,
        grid=(1,),
        in_specs=[
            pl.BlockSpec(memory_space=pl.ANY),
            pl.BlockSpec((1, H, D), lambda i: (0, 0, 0)),
            pl.BlockSpec((1, W, D), lambda i: (0, 0, 0)),
        ],
        out_specs=pl.BlockSpec(memory_space=pl.ANY),
        out_shape=jax.ShapeDtypeStruct((B, H, W, D), x.dtype),
        scratch_shapes=[
            pltpu.VMEM((H, W, D), x.dtype),
            pltpu.VMEM((_NBUF, _CHUNK, H, W, D), x.dtype),
            pltpu.SemaphoreType.DMA((_NBUF,)),
            pltpu.SemaphoreType.DMA((_NBUF,)),
        ],
    )(x, h_table[None], w_table[None])
